# baseline (speedup 1.0000x reference)
import jax
import jax.numpy as jnp
from jax import lax
from jax.experimental import pallas as pl
from jax.experimental.pallas import tpu as pltpu
from jax.experimental.pallas import tpu_sc as plsc

N = 10000
E = 320000
D = 128

NC = 2
NS = 16
NW = NC * NS
ET = E // 128
WT = ET // NW
EPW = WT * 128
NX = ET - NW * WT
IW = WT * 256
XBW = NW * IW
CH = 128
GCH = 64
GFULL = 2 * WT
ZCH = 624
ZTAIL = N - NS * ZCH
ZBLOCKS = [(k * GCH, GCH) for k in range(ZCH // GCH)]
if ZCH % GCH:
    ZBLOCKS.append(((ZCH // GCH) * GCH, ZCH % GCH))

_MESH = plsc.VectorSubcoreMesh(core_axis_name="c", subcore_axis_name="s")


def _deg_body(z_hbm, hist_hbm, ones_v, zeros_v, idx_all, eidx, hist_sh, ssem):
    cid = lax.axis_index("c")
    sid = lax.axis_index("s")
    wid = sid * NC + cid

    one16 = jnp.full((16,), 1.0, jnp.float32)
    zero16 = jnp.zeros((16,), jnp.float32)

    def fill_ones(i, c):
        ones_v[pl.ds(i * 16, 16)] = one16
        return c

    lax.fori_loop(0, CH // 16, fill_ones, 0)

    def fill_zeros(i, c):
        zeros_v[pl.ds(i * 16, 16)] = zero16
        return c

    lax.fori_loop(0, ZCH // 16, fill_zeros, 0)

    pltpu.sync_copy(z_hbm.at[pl.ds(wid * IW, IW)], idx_all)

    pltpu.sync_copy(zeros_v, hist_sh.at[pl.ds(sid * ZCH, ZCH)])

    @pl.when(sid == 0)
    def _():
        pltpu.sync_copy(zeros_v.at[pl.ds(0, ZTAIL)],
                        hist_sh.at[pl.ds(NS * ZCH, ZTAIL)])

    plsc.subcore_barrier()

    def fire(j, c):
        pltpu.async_copy(ones_v, hist_sh.at[idx_all.at[pl.ds(j * 256, CH)]],
                         ssem, add=True)
        return c

    lax.fori_loop(0, WT, fire, 0)

    @pl.when(wid >= NW - NX)
    def _():
        xb = XBW + (wid - (NW - NX)) * 256
        pltpu.sync_copy(z_hbm.at[pl.ds(xb, 256)], eidx)
        pltpu.sync_copy(ones_v, hist_sh.at[eidx.at[pl.ds(0, CH)]], add=True)

    def drain(j, c):
        pltpu.make_async_copy(ones_v, hist_sh.at[idx_all.at[pl.ds(0, CH)]],
                              ssem).wait()
        return c

    lax.fori_loop(0, WT, drain, 0)

    plsc.subcore_barrier()

    hbase = cid * N
    pltpu.sync_copy(hist_sh.at[pl.ds(sid * ZCH, ZCH)], zeros_v)
    pltpu.sync_copy(zeros_v, hist_hbm.at[pl.ds(hbase + sid * ZCH, ZCH)])

    @pl.when(sid == 0)
    def _():
        pltpu.sync_copy(hist_sh.at[pl.ds(NS * ZCH, ZTAIL)],
                        ones_v.at[pl.ds(0, ZTAIL)])
        pltpu.sync_copy(ones_v.at[pl.ds(0, ZTAIL)],
                        hist_hbm.at[pl.ds(hbase + NS * ZCH, ZTAIL)])


_deg_hist = pl.kernel(
    _deg_body,
    out_type=jax.ShapeDtypeStruct((NC * N,), jnp.float32),
    mesh=_MESH,
    scratch_types=[
        pltpu.VMEM((CH,), jnp.float32),
        pltpu.VMEM((ZCH,), jnp.float32),
        pltpu.VMEM((IW,), jnp.int32),
        pltpu.VMEM((256,), jnp.int32),
        pltpu.VMEM_SHARED((N,), jnp.float32),
        pltpu.SemaphoreType.DMA,
    ],
)


NBUF = 3
NGRP = GFULL // NBUF


def _scatter_body(y_hbm, z_hbm, part_hbm,
                  idx_all, eidx, gbuf0, gbuf1, gbuf2,
                  gsem0, gsem1, gsem2, ssem0, ssem1, ssem2, tsem, acc_sh):
    cid = lax.axis_index("c")
    sid = lax.axis_index("s")
    wid = sid * NC + cid
    gbufs = [gbuf0, gbuf1, gbuf2]
    gsems = [gsem0, gsem1, gsem2]
    ssems = [ssem0, ssem1, ssem2]

    pltpu.async_copy(z_hbm.at[pl.ds(wid * IW, IW)], idx_all, tsem)

    zero16 = jnp.zeros((16,), jnp.float32)

    def zrow(r, c):
        for k in range(D // 16):
            gbuf2[r, pl.ds(k * 16, 16)] = zero16
        return c

    lax.fori_loop(0, GCH, zrow, 0)

    pltpu.make_async_copy(z_hbm.at[pl.ds(wid * IW, IW)], idx_all, tsem).wait()

    def cidx(cj):
        return idx_all.at[pl.ds((cj // 2) * 256 + 128 + (cj % 2) * GCH, GCH)]

    def ridx(cj):
        return idx_all.at[pl.ds((cj // 2) * 256 + (cj % 2) * GCH, GCH)]

    for b in range(NBUF - 1):
        pltpu.async_copy(y_hbm.at[cidx(b)], gbufs[b], gsems[b])

    rbase = sid * ZCH
    for off, sz in ZBLOCKS:
        pltpu.sync_copy(gbuf2.at[pl.ds(0, sz)], acc_sh.at[pl.ds(rbase + off, sz)])

    @pl.when(sid == 0)
    def _():
        pltpu.sync_copy(gbuf2.at[pl.ds(0, ZTAIL)],
                        acc_sh.at[pl.ds(NS * ZCH, ZTAIL)])

    pltpu.async_copy(y_hbm.at[cidx(NBUF - 1)], gbufs[NBUF - 1], gsems[NBUF - 1])

    plsc.subcore_barrier()

    def grp(j, c):
        for b in range(NBUF):
            cj = j * NBUF + b
            pltpu.make_async_copy(y_hbm.at[cidx(cj)], gbufs[b], gsems[b]).wait()
            pltpu.async_copy(gbufs[b], acc_sh.at[ridx(cj)], ssems[b], add=True)
            pltpu.make_async_copy(gbufs[b], acc_sh.at[ridx(cj)], ssems[b]).wait()
            pltpu.async_copy(y_hbm.at[cidx(cj + NBUF)], gbufs[b], gsems[b])
        return c

    lax.fori_loop(0, NGRP - 1, grp, 0)

    for b in range(NBUF):
        cj = (NGRP - 1) * NBUF + b
        pltpu.make_async_copy(y_hbm.at[cidx(cj)], gbufs[b], gsems[b]).wait()
        pltpu.async_copy(gbufs[b], acc_sh.at[ridx(cj)], ssems[b], add=True)
        pltpu.make_async_copy(gbufs[b], acc_sh.at[ridx(cj)], ssems[b]).wait()

    @pl.when(wid >= NW - NX)
    def _():
        xb = XBW + (wid - (NW - NX)) * 256
        pltpu.sync_copy(z_hbm.at[pl.ds(xb, 256)], eidx)
        for b in range(2):
            pltpu.async_copy(y_hbm.at[eidx.at[pl.ds(128 + b * GCH, GCH)]],
                             gbufs[b], tsem).wait()
            pltpu.sync_copy(gbufs[b],
                            acc_sh.at[eidx.at[pl.ds(b * GCH, GCH)]], add=True)

    plsc.subcore_barrier()

    pbase = cid * N + rbase
    wbufs = (gbuf0, gbuf1)
    for k, (off, sz) in enumerate(ZBLOCKS):
        b = k % 2
        if k >= 2:
            psz = ZBLOCKS[k - 2][1]
            pltpu.make_async_copy(wbufs[b].at[pl.ds(0, psz)],
                                  part_hbm.at[pl.ds(pbase, psz)],
                                  ssems[b]).wait()
        pltpu.async_copy(acc_sh.at[pl.ds(rbase + off, sz)],
                         wbufs[b].at[pl.ds(0, sz)], gsems[b])
        pltpu.make_async_copy(acc_sh.at[pl.ds(rbase + off, sz)],
                              wbufs[b].at[pl.ds(0, sz)], gsems[b]).wait()
        pltpu.async_copy(wbufs[b].at[pl.ds(0, sz)],
                         part_hbm.at[pl.ds(pbase + off, sz)], ssems[b])
    nb = len(ZBLOCKS)
    for k in (nb - 2, nb - 1):
        b = k % 2
        psz = ZBLOCKS[k][1]
        pltpu.make_async_copy(wbufs[b].at[pl.ds(0, psz)],
                              part_hbm.at[pl.ds(pbase, psz)], ssems[b]).wait()

    @pl.when(sid == 0)
    def _():
        pltpu.sync_copy(acc_sh.at[pl.ds(NS * ZCH, ZTAIL)],
                        gbuf2.at[pl.ds(0, ZTAIL)])
        pltpu.sync_copy(gbuf2.at[pl.ds(0, ZTAIL)],
                        part_hbm.at[pl.ds(cid * N + NS * ZCH, ZTAIL)])


_edge_scatter = pl.kernel(
    _scatter_body,
    out_type=jax.ShapeDtypeStruct((NC * N, D), jnp.float32),
    mesh=_MESH,
    scratch_types=[
        pltpu.VMEM((IW,), jnp.int32),
        pltpu.VMEM((256,), jnp.int32),
        pltpu.VMEM((GCH, D), jnp.float32),
        pltpu.VMEM((GCH, D), jnp.float32),
        pltpu.VMEM((GCH, D), jnp.float32),
        pltpu.SemaphoreType.DMA,
        pltpu.SemaphoreType.DMA,
        pltpu.SemaphoreType.DMA,
        pltpu.SemaphoreType.DMA,
        pltpu.SemaphoreType.DMA,
        pltpu.SemaphoreType.DMA,
        pltpu.SemaphoreType.DMA,
        pltpu.VMEM_SHARED((N, D), jnp.float32),
    ],
)


def _mm_body(x_ref, w_ref, xw_ref):
    xw_ref[...] = lax.dot_general(x_ref[...], w_ref[...],
                                  (((1,), (1,)), ((), ())),
                                  preferred_element_type=jnp.float32)


_mm = pl.pallas_call(
    _mm_body,
    out_shape=jax.ShapeDtypeStruct((N, D), jnp.float32),
)


def _scale_body(hist_ref, xw_ref, y_ref, dis_ref):
    deg = hist_ref[pl.ds(0, N)] + hist_ref[pl.ds(N, N)] + 1.0
    dis = lax.rsqrt(deg)
    dis_ref[...] = dis
    y_ref[...] = xw_ref[...] * dis[:, None]


_scale = pl.pallas_call(
    _scale_body,
    out_shape=(jax.ShapeDtypeStruct((N, D), jnp.float32),
               jax.ShapeDtypeStruct((N,), jnp.float32)),
)


def _fin_body(p_ref, y_ref, dis_ref, o_ref):
    o_ref[...] = ((p_ref[pl.ds(0, N), :] + p_ref[pl.ds(N, N), :] + y_ref[...])
                  * dis_ref[...][:, None])


_fin = pl.pallas_call(
    _fin_body,
    out_shape=jax.ShapeDtypeStruct((N, D), jnp.float32),
)


@jax.jit
def kernel(x, edge_index, W):
    x = x.astype(jnp.float32)
    W = W.astype(jnp.float32)
    ei = edge_index.astype(jnp.int32)
    z = jnp.transpose(ei.reshape(2, ET, 128), (1, 0, 2)).reshape(2 * E)
    hist = _deg_hist(z)
    xw = _mm(x, W)
    y, dis = _scale(hist, xw)
    part = _edge_scatter(y, z)
    return _fin(part, y, dis)


# speedup vs baseline: 1.1007x; 1.1007x over previous
import functools

import jax
import jax.numpy as jnp
from jax import lax
from jax.experimental import pallas as pl
from jax.experimental.pallas import tpu as pltpu
from jax.experimental.pallas import tpu_sc as plsc

N = 10000
E = 320000
D = 128

NC = 2
NS = 16
NW = NC * NS
EPW = E // NW
CH = 128
NFULL = EPW // CH
TAIL = EPW - NFULL * CH
GCH = 48
GFULL = EPW // GCH
GTAIL = EPW - GFULL * GCH
ZCH = 624
ZTAIL = N - NS * ZCH

_MESH = plsc.VectorSubcoreMesh(core_axis_name="c", subcore_axis_name="s")


def _deg_body(rc_hbm, hist_hbm, ones_v, zeros_v, idx_all, ones_t, hist_sh, ssem):
    cid = lax.axis_index("c")
    sid = lax.axis_index("s")
    wid = sid * NC + cid

    one16 = jnp.full((16,), 1.0, jnp.float32)
    zero16 = jnp.zeros((16,), jnp.float32)

    def fill_ones(i, c):
        ones_v[pl.ds(i * 16, 16)] = one16
        return c

    lax.fori_loop(0, CH // 16, fill_ones, 0)
    ones_t[...] = one16

    def fill_zeros(i, c):
        zeros_v[pl.ds(i * 16, 16)] = zero16
        return c

    lax.fori_loop(0, ZCH // 16, fill_zeros, 0)

    base = wid * EPW
    pltpu.sync_copy(rc_hbm.at[pl.ds(base, EPW)], idx_all)

    pltpu.sync_copy(zeros_v, hist_sh.at[pl.ds(sid * ZCH, ZCH)])

    @pl.when(sid == 0)
    def _():
        pltpu.sync_copy(zeros_v.at[pl.ds(0, ZTAIL)],
                        hist_sh.at[pl.ds(NS * ZCH, ZTAIL)])

    plsc.subcore_barrier()

    def fire(j, c):
        pltpu.async_copy(ones_v, hist_sh.at[idx_all.at[pl.ds(j * CH, CH)]],
                         ssem, add=True)
        return c

    lax.fori_loop(0, NFULL, fire, 0)
    pltpu.sync_copy(ones_t, hist_sh.at[idx_all.at[pl.ds(NFULL * CH, TAIL)]],
                    add=True)

    def drain(j, c):
        pltpu.make_async_copy(ones_v, hist_sh.at[idx_all.at[pl.ds(0, CH)]],
                              ssem).wait()
        return c

    lax.fori_loop(0, NFULL, drain, 0)

    plsc.subcore_barrier()

    hbase = cid * N
    pltpu.sync_copy(hist_sh.at[pl.ds(sid * ZCH, ZCH)], zeros_v)
    pltpu.sync_copy(zeros_v, hist_hbm.at[pl.ds(hbase + sid * ZCH, ZCH)])

    @pl.when(sid == 0)
    def _():
        pltpu.sync_copy(hist_sh.at[pl.ds(NS * ZCH, ZTAIL)], ones_t)
        pltpu.sync_copy(ones_t, hist_hbm.at[pl.ds(hbase + NS * ZCH, ZTAIL)])


_deg_hist = pl.kernel(
    _deg_body,
    out_type=jax.ShapeDtypeStruct((NC * N,), jnp.float32),
    mesh=_MESH,
    scratch_types=[
        pltpu.VMEM((CH,), jnp.float32),
        pltpu.VMEM((ZCH,), jnp.float32),
        pltpu.VMEM((EPW,), jnp.int32),
        pltpu.VMEM((TAIL,), jnp.float32),
        pltpu.VMEM_SHARED((N,), jnp.float32),
        pltpu.SemaphoreType.DMA,
    ],
)


NBUF = 4
NGRP = GFULL // NBUF


def _scatter_body(y_hbm, rc_hbm, part_hbm,
                  cidx_all, ridx_all, gbuf0, gbuf1, gbuf2, gbuf3,
                  gsem0, gsem1, gsem2, gsem3,
                  ssem0, ssem1, ssem2, ssem3, tsem, acc_sh):
    cid = lax.axis_index("c")
    sid = lax.axis_index("s")
    wid = sid * NC + cid
    gbufs = [gbuf0, gbuf1, gbuf2, gbuf3]
    gsems = [gsem0, gsem1, gsem2, gsem3]
    ssems = [ssem0, ssem1, ssem2, ssem3]

    base = wid * EPW
    pltpu.async_copy(rc_hbm.at[pl.ds(E + base, EPW)], cidx_all, tsem)
    pltpu.async_copy(rc_hbm.at[pl.ds(base, EPW)], ridx_all, tsem)

    zero16 = jnp.zeros((16,), jnp.float32)

    def zrow(r, c):
        for k in range(D // 16):
            gbuf3[r, pl.ds(k * 16, 16)] = zero16
        return c

    lax.fori_loop(0, GCH, zrow, 0)

    pltpu.make_async_copy(rc_hbm.at[pl.ds(E + base, EPW)], cidx_all, tsem).wait()
    pltpu.make_async_copy(rc_hbm.at[pl.ds(base, EPW)], ridx_all, tsem).wait()

    def cidx(cj):
        return cidx_all.at[pl.ds(cj * GCH, GCH)]

    def ridx(cj):
        return ridx_all.at[pl.ds(cj * GCH, GCH)]

    for b in range(NBUF - 1):
        pltpu.async_copy(y_hbm.at[cidx(b)], gbufs[b], gsems[b])

    rbase = sid * ZCH
    for k in range(ZCH // GCH):
        pltpu.sync_copy(gbuf3, acc_sh.at[pl.ds(rbase + k * GCH, GCH)])
    if ZCH % GCH:
        pltpu.sync_copy(gbuf3.at[pl.ds(0, ZCH % GCH)],
                        acc_sh.at[pl.ds(rbase + (ZCH // GCH) * GCH, ZCH % GCH)])

    @pl.when(sid == 0)
    def _():
        pltpu.sync_copy(gbuf3.at[pl.ds(0, ZTAIL)], acc_sh.at[pl.ds(NS * ZCH, ZTAIL)])

    pltpu.async_copy(y_hbm.at[cidx(NBUF - 1)], gbufs[NBUF - 1], gsems[NBUF - 1])

    plsc.subcore_barrier()

    def grp(j, c):
        for b in range(NBUF):
            cj = j * NBUF + b
            pltpu.make_async_copy(y_hbm.at[cidx(cj)], gbufs[b], gsems[b]).wait()
            pltpu.async_copy(gbufs[b], acc_sh.at[ridx(cj)], ssems[b], add=True)
            pltpu.make_async_copy(gbufs[b], acc_sh.at[ridx(cj)], ssems[b]).wait()
            pltpu.async_copy(y_hbm.at[cidx(cj + NBUF)], gbufs[b], gsems[b])
        return c

    lax.fori_loop(0, NGRP - 1, grp, 0)

    for b in range(NBUF):
        cj = (NGRP - 1) * NBUF + b
        pltpu.make_async_copy(y_hbm.at[cidx(cj)], gbufs[b], gsems[b]).wait()
        pltpu.async_copy(gbufs[b], acc_sh.at[ridx(cj)], ssems[b], add=True)
        pltpu.make_async_copy(gbufs[b], acc_sh.at[ridx(cj)], ssems[b]).wait()

    off = GFULL * GCH
    pltpu.async_copy(y_hbm.at[cidx_all.at[pl.ds(off, GTAIL)]],
                     gbuf0.at[pl.ds(0, GTAIL)], tsem).wait()
    pltpu.sync_copy(gbuf0.at[pl.ds(0, GTAIL)],
                    acc_sh.at[ridx_all.at[pl.ds(off, GTAIL)]], add=True)

    plsc.subcore_barrier()

    pbase = cid * N + rbase
    wbufs = (gbuf0, gbuf1)
    for k in range(ZCH // GCH):
        b = k % 2
        if k >= 2:
            pltpu.make_async_copy(wbufs[b], part_hbm.at[pl.ds(pbase, GCH)],
                                  ssems[b]).wait()
        pltpu.async_copy(acc_sh.at[pl.ds(rbase + k * GCH, GCH)], wbufs[b],
                         gsems[b])
        pltpu.make_async_copy(acc_sh.at[pl.ds(rbase + k * GCH, GCH)], wbufs[b],
                              gsems[b]).wait()
        pltpu.async_copy(wbufs[b], part_hbm.at[pl.ds(pbase + k * GCH, GCH)],
                         ssems[b])
    for b in range(2):
        pltpu.make_async_copy(wbufs[b], part_hbm.at[pl.ds(pbase, GCH)],
                              ssems[b]).wait()

    @pl.when(sid == 0)
    def _():
        pltpu.sync_copy(acc_sh.at[pl.ds(NS * ZCH, ZTAIL)], gbuf1.at[pl.ds(0, ZTAIL)])
        pltpu.sync_copy(gbuf1.at[pl.ds(0, ZTAIL)],
                        part_hbm.at[pl.ds(cid * N + NS * ZCH, ZTAIL)])


_edge_scatter = pl.kernel(
    _scatter_body,
    out_type=jax.ShapeDtypeStruct((NC * N, D), jnp.float32),
    mesh=_MESH,
    scratch_types=[
        pltpu.VMEM((EPW,), jnp.int32),
        pltpu.VMEM((EPW,), jnp.int32),
        pltpu.VMEM((GCH, D), jnp.float32),
        pltpu.VMEM((GCH, D), jnp.float32),
        pltpu.VMEM((GCH, D), jnp.float32),
        pltpu.VMEM((GCH, D), jnp.float32),
        pltpu.SemaphoreType.DMA,
        pltpu.SemaphoreType.DMA,
        pltpu.SemaphoreType.DMA,
        pltpu.SemaphoreType.DMA,
        pltpu.SemaphoreType.DMA,
        pltpu.SemaphoreType.DMA,
        pltpu.SemaphoreType.DMA,
        pltpu.SemaphoreType.DMA,
        pltpu.SemaphoreType.DMA,
        pltpu.VMEM_SHARED((N, D), jnp.float32),
    ],
)


def _cvt_body(ei_ref, rc_ref):
    rc_ref[pl.ds(0, E)] = ei_ref[0, :]
    rc_ref[pl.ds(E, E)] = ei_ref[1, :]


_cvt = pl.pallas_call(
    _cvt_body,
    out_shape=jax.ShapeDtypeStruct((2 * E,), jnp.int32),
)


def _mm_body(x_ref, w_ref, xw_ref):
    xw_ref[...] = lax.dot_general(x_ref[...], w_ref[...],
                                  (((1,), (1,)), ((), ())),
                                  preferred_element_type=jnp.float32)


_mm = pl.pallas_call(
    _mm_body,
    out_shape=jax.ShapeDtypeStruct((N, D), jnp.float32),
)


def _scale_body(hist_ref, xw_ref, y_ref, dis_ref):
    deg = hist_ref[pl.ds(0, N)] + hist_ref[pl.ds(N, N)] + 1.0
    dis = lax.rsqrt(deg)
    dis_ref[...] = dis
    y_ref[...] = xw_ref[...] * dis[:, None]


_scale = pl.pallas_call(
    _scale_body,
    out_shape=(jax.ShapeDtypeStruct((N, D), jnp.float32),
               jax.ShapeDtypeStruct((N,), jnp.float32)),
)


def _fin_body(p_ref, y_ref, dis_ref, o_ref):
    o_ref[...] = ((p_ref[pl.ds(0, N), :] + p_ref[pl.ds(N, N), :] + y_ref[...])
                  * dis_ref[...][:, None])


_fin = pl.pallas_call(
    _fin_body,
    out_shape=jax.ShapeDtypeStruct((N, D), jnp.float32),
)


@jax.jit
def kernel(x, edge_index, W):
    x = x.astype(jnp.float32)
    W = W.astype(jnp.float32)
    rc = _cvt(edge_index.astype(jnp.int32))
    hist = _deg_hist(rc)
    xw = _mm(x, W)
    y, dis = _scale(hist, xw)
    part = _edge_scatter(y, rc)
    return _fin(part, y, dis)


# speedup vs baseline: 1.1178x; 1.0156x over previous
import functools

import jax
import jax.numpy as jnp
from jax import lax
from jax.experimental import pallas as pl
from jax.experimental.pallas import tpu as pltpu
from jax.experimental.pallas import tpu_sc as plsc

N = 10000
E = 320000
D = 128

NC = 2
NS = 16
NW = NC * NS
EPW = E // NW
CH = 128
NFULL = EPW // CH
TAIL = EPW - NFULL * CH
GCH = 40
GFULL = EPW // GCH
GTAIL = EPW - GFULL * GCH
ZCH = 624
ZTAIL = N - NS * ZCH

_MESH = plsc.VectorSubcoreMesh(core_axis_name="c", subcore_axis_name="s")


def _deg_body(rc_hbm, hist_hbm, ones_v, zeros_v, idx_all, ones_t, hist_sh, ssem):
    cid = lax.axis_index("c")
    sid = lax.axis_index("s")
    wid = sid * NC + cid

    one16 = jnp.full((16,), 1.0, jnp.float32)
    zero16 = jnp.zeros((16,), jnp.float32)

    def fill_ones(i, c):
        ones_v[pl.ds(i * 16, 16)] = one16
        return c

    lax.fori_loop(0, CH // 16, fill_ones, 0)
    ones_t[...] = one16

    def fill_zeros(i, c):
        zeros_v[pl.ds(i * 16, 16)] = zero16
        return c

    lax.fori_loop(0, ZCH // 16, fill_zeros, 0)

    base = wid * EPW
    pltpu.sync_copy(rc_hbm.at[pl.ds(base, EPW)], idx_all)

    pltpu.sync_copy(zeros_v, hist_sh.at[pl.ds(sid * ZCH, ZCH)])

    @pl.when(sid == 0)
    def _():
        pltpu.sync_copy(zeros_v.at[pl.ds(0, ZTAIL)],
                        hist_sh.at[pl.ds(NS * ZCH, ZTAIL)])

    plsc.subcore_barrier()

    def fire(j, c):
        pltpu.async_copy(ones_v, hist_sh.at[idx_all.at[pl.ds(j * CH, CH)]],
                         ssem, add=True)
        return c

    lax.fori_loop(0, NFULL, fire, 0)
    pltpu.sync_copy(ones_t, hist_sh.at[idx_all.at[pl.ds(NFULL * CH, TAIL)]],
                    add=True)

    def drain(j, c):
        pltpu.make_async_copy(ones_v, hist_sh.at[idx_all.at[pl.ds(0, CH)]],
                              ssem).wait()
        return c

    lax.fori_loop(0, NFULL, drain, 0)

    plsc.subcore_barrier()

    hbase = cid * N
    pltpu.sync_copy(hist_sh.at[pl.ds(sid * ZCH, ZCH)], zeros_v)
    pltpu.sync_copy(zeros_v, hist_hbm.at[pl.ds(hbase + sid * ZCH, ZCH)])

    @pl.when(sid == 0)
    def _():
        pltpu.sync_copy(hist_sh.at[pl.ds(NS * ZCH, ZTAIL)], ones_t)
        pltpu.sync_copy(ones_t, hist_hbm.at[pl.ds(hbase + NS * ZCH, ZTAIL)])


_deg_hist = pl.kernel(
    _deg_body,
    out_type=jax.ShapeDtypeStruct((NC * N,), jnp.float32),
    mesh=_MESH,
    scratch_types=[
        pltpu.VMEM((CH,), jnp.float32),
        pltpu.VMEM((ZCH,), jnp.float32),
        pltpu.VMEM((EPW,), jnp.int32),
        pltpu.VMEM((TAIL,), jnp.float32),
        pltpu.VMEM_SHARED((N,), jnp.float32),
        pltpu.SemaphoreType.DMA,
    ],
)


NBUF = 5
NGRP = GFULL // NBUF


def _scatter_body(y_hbm, rc_hbm, part_hbm,
                  cidx_all, ridx_all, gbuf0, gbuf1, gbuf2, gbuf3, gbuf4,
                  gsem0, gsem1, gsem2, gsem3, gsem4,
                  ssem0, ssem1, ssem2, ssem3, ssem4, tsem, acc_sh):
    cid = lax.axis_index("c")
    sid = lax.axis_index("s")
    wid = sid * NC + cid
    gbufs = [gbuf0, gbuf1, gbuf2, gbuf3, gbuf4]
    gsems = [gsem0, gsem1, gsem2, gsem3, gsem4]
    ssems = [ssem0, ssem1, ssem2, ssem3, ssem4]

    base = wid * EPW
    pltpu.async_copy(rc_hbm.at[pl.ds(E + base, EPW)], cidx_all, tsem)
    pltpu.async_copy(rc_hbm.at[pl.ds(base, EPW)], ridx_all, tsem)

    zero16 = jnp.zeros((16,), jnp.float32)

    def zrow(r, c):
        for k in range(D // 16):
            gbuf4[r, pl.ds(k * 16, 16)] = zero16
        return c

    lax.fori_loop(0, GCH, zrow, 0)

    pltpu.make_async_copy(rc_hbm.at[pl.ds(E + base, EPW)], cidx_all, tsem).wait()
    pltpu.make_async_copy(rc_hbm.at[pl.ds(base, EPW)], ridx_all, tsem).wait()

    def cidx(cj):
        return cidx_all.at[pl.ds(cj * GCH, GCH)]

    def ridx(cj):
        return ridx_all.at[pl.ds(cj * GCH, GCH)]

    for b in range(NBUF - 1):
        pltpu.async_copy(y_hbm.at[cidx(b)], gbufs[b], gsems[b])

    rbase = sid * ZCH
    for k in range(ZCH // GCH):
        pltpu.sync_copy(gbuf4, acc_sh.at[pl.ds(rbase + k * GCH, GCH)])
    if ZCH % GCH:
        pltpu.sync_copy(gbuf4.at[pl.ds(0, ZCH % GCH)],
                        acc_sh.at[pl.ds(rbase + (ZCH // GCH) * GCH, ZCH % GCH)])

    @pl.when(sid == 0)
    def _():
        pltpu.sync_copy(gbuf4.at[pl.ds(0, ZTAIL)], acc_sh.at[pl.ds(NS * ZCH, ZTAIL)])

    pltpu.async_copy(y_hbm.at[cidx(NBUF - 1)], gbufs[NBUF - 1], gsems[NBUF - 1])

    plsc.subcore_barrier()

    def grp(j, c):
        for b in range(NBUF):
            cj = j * NBUF + b
            pltpu.make_async_copy(y_hbm.at[cidx(cj)], gbufs[b], gsems[b]).wait()
            pltpu.async_copy(gbufs[b], acc_sh.at[ridx(cj)], ssems[b], add=True)
            pltpu.make_async_copy(gbufs[b], acc_sh.at[ridx(cj)], ssems[b]).wait()
            pltpu.async_copy(y_hbm.at[cidx(cj + NBUF)], gbufs[b], gsems[b])
        return c

    lax.fori_loop(0, NGRP - 1, grp, 0)

    for b in range(NBUF):
        cj = (NGRP - 1) * NBUF + b
        pltpu.make_async_copy(y_hbm.at[cidx(cj)], gbufs[b], gsems[b]).wait()
        pltpu.async_copy(gbufs[b], acc_sh.at[ridx(cj)], ssems[b], add=True)
        pltpu.make_async_copy(gbufs[b], acc_sh.at[ridx(cj)], ssems[b]).wait()

    if GTAIL:
        off = GFULL * GCH
        pltpu.async_copy(y_hbm.at[cidx_all.at[pl.ds(off, GTAIL)]],
                         gbuf0.at[pl.ds(0, GTAIL)], tsem).wait()
        pltpu.sync_copy(gbuf0.at[pl.ds(0, GTAIL)],
                        acc_sh.at[ridx_all.at[pl.ds(off, GTAIL)]], add=True)

    plsc.subcore_barrier()

    pbase = cid * N + rbase
    wbufs = (gbuf0, gbuf1)
    for k in range(ZCH // GCH):
        b = k % 2
        if k >= 2:
            pltpu.make_async_copy(wbufs[b], part_hbm.at[pl.ds(pbase, GCH)],
                                  ssems[b]).wait()
        pltpu.async_copy(acc_sh.at[pl.ds(rbase + k * GCH, GCH)], wbufs[b],
                         gsems[b])
        pltpu.make_async_copy(acc_sh.at[pl.ds(rbase + k * GCH, GCH)], wbufs[b],
                              gsems[b]).wait()
        pltpu.async_copy(wbufs[b], part_hbm.at[pl.ds(pbase + k * GCH, GCH)],
                         ssems[b])
    for b in range(2):
        pltpu.make_async_copy(wbufs[b], part_hbm.at[pl.ds(pbase, GCH)],
                              ssems[b]).wait()
    if ZCH % GCH:
        rem = ZCH % GCH
        roff = (ZCH // GCH) * GCH
        pltpu.sync_copy(acc_sh.at[pl.ds(rbase + roff, rem)],
                        gbuf0.at[pl.ds(0, rem)])
        pltpu.sync_copy(gbuf0.at[pl.ds(0, rem)],
                        part_hbm.at[pl.ds(pbase + roff, rem)])

    @pl.when(sid == 0)
    def _():
        pltpu.sync_copy(acc_sh.at[pl.ds(NS * ZCH, ZTAIL)], gbuf1.at[pl.ds(0, ZTAIL)])
        pltpu.sync_copy(gbuf1.at[pl.ds(0, ZTAIL)],
                        part_hbm.at[pl.ds(cid * N + NS * ZCH, ZTAIL)])


_edge_scatter = pl.kernel(
    _scatter_body,
    out_type=jax.ShapeDtypeStruct((NC * N, D), jnp.float32),
    mesh=_MESH,
    scratch_types=[
        pltpu.VMEM((EPW,), jnp.int32),
        pltpu.VMEM((EPW,), jnp.int32),
        pltpu.VMEM((GCH, D), jnp.float32),
        pltpu.VMEM((GCH, D), jnp.float32),
        pltpu.VMEM((GCH, D), jnp.float32),
        pltpu.VMEM((GCH, D), jnp.float32),
        pltpu.VMEM((GCH, D), jnp.float32),
        pltpu.SemaphoreType.DMA,
        pltpu.SemaphoreType.DMA,
        pltpu.SemaphoreType.DMA,
        pltpu.SemaphoreType.DMA,
        pltpu.SemaphoreType.DMA,
        pltpu.SemaphoreType.DMA,
        pltpu.SemaphoreType.DMA,
        pltpu.SemaphoreType.DMA,
        pltpu.SemaphoreType.DMA,
        pltpu.SemaphoreType.DMA,
        pltpu.SemaphoreType.DMA,
        pltpu.VMEM_SHARED((N, D), jnp.float32),
    ],
)


def _cvt_body(ei_ref, rc_ref):
    rc_ref[pl.ds(0, E)] = ei_ref[0, :]
    rc_ref[pl.ds(E, E)] = ei_ref[1, :]


_cvt = pl.pallas_call(
    _cvt_body,
    out_shape=jax.ShapeDtypeStruct((2 * E,), jnp.int32),
)


def _mm_body(x_ref, w_ref, xw_ref):
    xw_ref[...] = lax.dot_general(x_ref[...], w_ref[...],
                                  (((1,), (1,)), ((), ())),
                                  preferred_element_type=jnp.float32)


_mm = pl.pallas_call(
    _mm_body,
    out_shape=jax.ShapeDtypeStruct((N, D), jnp.float32),
)


def _scale_body(hist_ref, xw_ref, y_ref, dis_ref):
    deg = hist_ref[pl.ds(0, N)] + hist_ref[pl.ds(N, N)] + 1.0
    dis = lax.rsqrt(deg)
    dis_ref[...] = dis
    y_ref[...] = xw_ref[...] * dis[:, None]


_scale = pl.pallas_call(
    _scale_body,
    out_shape=(jax.ShapeDtypeStruct((N, D), jnp.float32),
               jax.ShapeDtypeStruct((N,), jnp.float32)),
)


def _fin_body(p_ref, y_ref, dis_ref, o_ref):
    o_ref[...] = ((p_ref[pl.ds(0, N), :] + p_ref[pl.ds(N, N), :] + y_ref[...])
                  * dis_ref[...][:, None])


_fin = pl.pallas_call(
    _fin_body,
    out_shape=jax.ShapeDtypeStruct((N, D), jnp.float32),
)


@jax.jit
def kernel(x, edge_index, W):
    x = x.astype(jnp.float32)
    W = W.astype(jnp.float32)
    rc = _cvt(edge_index.astype(jnp.int32))
    hist = _deg_hist(rc)
    xw = _mm(x, W)
    y, dis = _scale(hist, xw)
    part = _edge_scatter(y, rc)
    return _fin(part, y, dis)
